# Initial kernel scaffold; baseline (speedup 1.0000x reference)
import jax
import jax.numpy as jnp
from jax.experimental import pallas as pl


def kernel(seq, met, table):
    raise NotImplementedError("write your pallas kernel here")


# speedup vs baseline: 21.3628x; 21.3628x over previous
import functools

import jax
import jax.numpy as jnp
from jax import lax
from jax.experimental import pallas as pl
from jax.experimental.pallas import tpu as pltpu
from jax.experimental.pallas import tpu_sc as plsc

NC = 2
NS = 16
NW = NC * NS
L = 16

VOCAB = 1000000
DIM = 64
MET_ROW = 5

B_SEQ = 4096
L_SEQ = 200
N = B_SEQ * L_SEQ
B_PER_W = N // NW
CHUNK = 512
GSZ = 128
N_CHUNKS = B_PER_W // CHUNK
QUARTERS = DIM // L


def _body(table_hbm, seq_hbm, met_hbm, out_hbm,
          seq_v, idx_v, s_v, rows_v, row5_v, sem):
    wid = lax.axis_index("s") * NC + lax.axis_index("c")
    base0 = wid * B_PER_W

    pltpu.sync_copy(table_hbm.at[pl.ds(MET_ROW, 1), :], row5_v)
    r5 = [row5_v[0, pl.ds(q * L, L)] for q in range(QUARTERS)]

    def chunk_body(ci, _):
        base = base0 + ci * CHUNK
        pltpu.sync_copy(seq_hbm.at[pl.ds(base, CHUNK)], seq_v)
        pltpu.sync_copy(met_hbm.at[pl.ds(base, CHUNK)], s_v)

        def pre(g, _):
            sv = seq_v[pl.ds(g * L, L)]
            mv = s_v[pl.ds(g * L, L)]
            keep = sv != 0
            seq_v[pl.ds(g * L, L)] = jnp.where(keep, sv, MET_ROW)
            s_v[pl.ds(g * L, L)] = jnp.where(keep, mv, -1.0)
            return 0

        lax.fori_loop(0, CHUNK // L, pre, 0, unroll=2)

        copies = [
            pltpu.async_copy(
                table_hbm.at[seq_v.at[pl.ds(t * GSZ, GSZ)]],
                rows_v.at[pl.ds(t * GSZ, GSZ)],
                sem,
            )
            for t in range(CHUNK // GSZ)
        ]
        for cp in copies:
            cp.wait()

        def rowfn(i, _):
            sb = plsc.load_gather(s_v, [jnp.full((L,), i, jnp.int32)])
            for q in range(QUARTERS):
                v = rows_v[i, pl.ds(q * L, L)]
                rows_v[i, pl.ds(q * L, L)] = v + sb * r5[q]
            return 0

        lax.fori_loop(0, CHUNK, rowfn, 0, unroll=4)

        pltpu.sync_copy(rows_v, out_hbm.at[pl.ds(base, CHUNK), :])
        return 0

    lax.fori_loop(0, N_CHUNKS, chunk_body, 0)


@jax.jit
def _run(table, seq_f, met_f):
    mesh = plsc.VectorSubcoreMesh(
        core_axis_name="c", subcore_axis_name="s",
        num_cores=NC, num_subcores=NS,
    )
    f = pl.kernel(
        _body,
        out_type=jax.ShapeDtypeStruct((N, DIM), jnp.float32),
        mesh=mesh,
        compiler_params=pltpu.CompilerParams(
            needs_layout_passes=False, use_tc_tiling_on_sc=False,
        ),
        scratch_types=[
            pltpu.VMEM((CHUNK,), jnp.int32),
            pltpu.VMEM((CHUNK,), jnp.int32),
            pltpu.VMEM((CHUNK,), jnp.float32),
            pltpu.VMEM((CHUNK, DIM), jnp.float32),
            pltpu.VMEM((1, DIM), jnp.float32),
            pltpu.SemaphoreType.DMA,
        ],
    )
    return f(table, seq_f, met_f)


def kernel(seq, met, table):
    seq_f = seq.reshape(N)
    met_f = met.reshape(N)
    out = _run(table, seq_f, met_f)
    return out.reshape(B_SEQ, L_SEQ, DIM)
